# Initial kernel scaffold; baseline (speedup 1.0000x reference)
import jax
import jax.numpy as jnp
from jax.experimental import pallas as pl


def kernel(x, W, b):
    raise NotImplementedError("write your pallas kernel here")


# speedup vs baseline: 212.5436x; 212.5436x over previous
import jax
import jax.numpy as jnp
from jax.experimental import pallas as pl
from jax.experimental.pallas import tpu as pltpu

N_NODES = 2048
N_IN = 512
N_OUT = 256
BATCH = 1024

BK = 128
NB = (N_NODES - N_IN) // BK
NSER = (N_NODES - N_OUT - N_IN) // BK
GRP = 8
NGRP = BK // GRP


def _dag_kernel(xT_ref, w_ref, wd_ref, b_ref, yT_ref, aT_ref, z_ref):
    t = pl.program_id(0)

    @pl.when(t == 0)
    def _init():
        aT_ref[0:N_IN, :] = xT_ref[...]
        aT_ref[N_IN:, :] = jnp.zeros((N_NODES - N_IN, BATCH), jnp.float32)

    z = jnp.dot(w_ref[...], aT_ref[...], preferred_element_type=jnp.float32)
    z_ref[...] = z + b_ref[0]

    s = N_IN + t * BK

    @pl.when(t < NSER)
    def _serial():
        wd = wd_ref[0]
        for g in range(NGRP):
            lo = g * GRP
            wd8 = wd[lo:lo + GRP, lo:lo + GRP]
            rows = []
            for i in range(GRP):
                r = lo + i
                arow = jnp.tanh(z_ref[r:r + 1, :])
                rows.append(arow)
                wcol = wd8[:, i:i + 1]
                z_ref[lo:lo + GRP, :] += wcol * arow
            a8 = jnp.concatenate(rows, axis=0)
            aT_ref[pl.ds(s + lo, GRP), :] = a8
            if g < NGRP - 1:
                z_ref[...] += jnp.dot(wd[:, lo:lo + GRP], a8,
                                      preferred_element_type=jnp.float32)

    @pl.when(t >= NSER)
    def _emit_output():
        yT_ref[...] = jnp.tanh(z_ref[...])


def kernel(x, W, b):
    xT = x.T
    diag = [
        jax.lax.slice(W, (N_IN + i * BK, N_IN + i * BK),
                      (N_IN + (i + 1) * BK, N_IN + (i + 1) * BK))
        for i in range(NSER)
    ]
    diag += [jnp.zeros((BK, BK), W.dtype)] * (NB - NSER)
    wdiag = jnp.stack(diag)
    b3 = b[N_IN:].reshape(NB, BK, 1)

    yT = pl.pallas_call(
        _dag_kernel,
        grid=(NB,),
        in_specs=[
            pl.BlockSpec((N_IN, BATCH), lambda t: (0, 0)),
            pl.BlockSpec((BK, N_NODES), lambda t: (N_IN // BK + t, 0)),
            pl.BlockSpec((1, BK, BK), lambda t: (t, 0, 0)),
            pl.BlockSpec((1, BK, 1), lambda t: (t, 0, 0)),
        ],
        out_specs=pl.BlockSpec((BK, BATCH), lambda t: (jnp.maximum(t - NSER, 0), 0)),
        out_shape=jax.ShapeDtypeStruct((N_OUT, BATCH), jnp.float32),
        scratch_shapes=[
            pltpu.VMEM((N_NODES, BATCH), jnp.float32),
            pltpu.VMEM((BK, BATCH), jnp.float32),
        ],
    )(xT, W, wdiag, b3)
    return yT.T


# speedup vs baseline: 415.8185x; 1.9564x over previous
import jax
import jax.numpy as jnp
from jax.experimental import pallas as pl
from jax.experimental.pallas import tpu as pltpu

N_NODES = 2048
N_IN = 512
N_OUT = 256
BATCH = 1024

BK = 128
NB = (N_NODES - N_IN) // BK
NSER = (N_NODES - N_OUT - N_IN) // BK
GRP = 8
NGRP = BK // GRP
KC = 256
NKC = N_NODES // KC


def _dag_kernel(xT_ref, w_ref, wd_ref, b_ref, yT_ref, aT_ref, z_ref):
    t = pl.program_id(0)

    @pl.when(t == 0)
    def _init():
        aT_ref[0:N_IN, :] = xT_ref[...]
        aT_ref[N_IN:, :] = jnp.zeros((N_NODES - N_IN, BATCH), jnp.float32)

    s = N_IN + t * BK
    kmax = jnp.minimum(s + BK, N_NODES - N_OUT)

    z_ref[...] = (
        jnp.dot(w_ref[:, 0:3 * KC], aT_ref[0:3 * KC, :],
                preferred_element_type=jnp.float32)
        + b_ref[0]
    )
    for c in range(3, NKC):
        @pl.when(KC * c < kmax)
        def _chunk(c=c):
            z_ref[...] += jnp.dot(w_ref[:, KC * c:KC * (c + 1)],
                                  aT_ref[KC * c:KC * (c + 1), :],
                                  preferred_element_type=jnp.float32)

    @pl.when(t < NSER)
    def _serial():
        zw = z_ref[0:GRP, :]
        for g in range(NGRP):
            lo = g * GRP
            last = g == NGRP - 1
            if not last:
                zw_next = z_ref[lo + GRP:lo + 2 * GRP, :]
                if g > 0:
                    zw_next += jnp.dot(
                        wd_ref[lo + GRP:lo + 2 * GRP, 0:lo],
                        aT_ref[pl.ds(s, lo), :],
                        preferred_element_type=jnp.float32)
                wd_win = wd_ref[lo:lo + 2 * GRP, lo:lo + GRP]
            else:
                wd_win = wd_ref[lo:lo + GRP, lo:lo + GRP]
            rows = []
            for i in range(GRP):
                arow = jnp.tanh(zw[i:i + 1, :])
                rows.append(arow)
                zw = zw + wd_win[0:GRP, i:i + 1] * arow
                if not last:
                    zw_next = zw_next + wd_win[GRP:2 * GRP, i:i + 1] * arow
            a8 = jnp.concatenate(rows, axis=0)
            aT_ref[pl.ds(s + lo, GRP), :] = a8
            if not last:
                zw = zw_next

    @pl.when(t >= NSER)
    def _emit_output():
        yT_ref[...] = jnp.tanh(z_ref[...])


def kernel(x, W, b):
    xT = x.T
    b3 = b[N_IN:].reshape(NB, BK, 1)
    hb = N_IN // BK

    yT = pl.pallas_call(
        _dag_kernel,
        grid=(NB,),
        in_specs=[
            pl.BlockSpec((N_IN, BATCH), lambda t: (0, 0)),
            pl.BlockSpec((BK, N_NODES), lambda t: (hb + t, 0)),
            pl.BlockSpec((BK, BK), lambda t: (hb + t, hb + t)),
            pl.BlockSpec((1, BK, 1), lambda t: (t, 0, 0)),
        ],
        out_specs=pl.BlockSpec((BK, BATCH), lambda t: (jnp.maximum(t - NSER, 0), 0)),
        out_shape=jax.ShapeDtypeStruct((N_OUT, BATCH), jnp.float32),
        scratch_shapes=[
            pltpu.VMEM((N_NODES, BATCH), jnp.float32),
            pltpu.VMEM((BK, BATCH), jnp.float32),
        ],
    )(xT, W, W, b3)
    return yT.T
